# baseline (speedup 1.0000x reference)
import functools

import jax
import jax.numpy as jnp
from jax import lax
from jax.experimental import pallas as pl
from jax.experimental.pallas import tpu as pltpu
from jax.experimental.pallas import tpu_sc as plsc

_T = 20
_NDT = 125
_NBT = 32
_BATCH = 4096
_U = _NDT * _T


def _build():
    info = plsc.get_sparse_core_info()
    nc = info.num_cores
    nw = nc * info.num_subcores
    mesh = plsc.VectorSubcoreMesh(core_axis_name="c", subcore_axis_name="s")

    @functools.partial(
        pl.kernel,
        mesh=mesh,
        out_type=jax.ShapeDtypeStruct((_T, _NDT, _NBT, 8, 128), jnp.float32),
        scratch_types=[
            pltpu.VMEM((2 * _BATCH,), jnp.int32),
            pltpu.VMEM((16000,), jnp.float32),
            pltpu.VMEM((2, 16, 8, 128), jnp.float32),
            pltpu.SemaphoreType.DMA,
            pltpu.SemaphoreType.DMA,
            pltpu.SemaphoreType.DMA,
        ],
        compiler_params=pltpu.CompilerParams(
            use_tc_tiling_on_sc=False, needs_layout_passes=False),
    )
    def emb(xt_hbm, tabr_hbm, out_hbm, x_v, slice_v, blk_v, xsem, ssem, osem):
        wid = lax.axis_index("s") * nc + lax.axis_index("c")
        lo = wid * _U // nw
        hi = (wid + 1) * _U // nw

        ds_off = [jnp.full((16,), ds * 1000, jnp.int32) for ds in range(8)]

        def stage_x(u):
            pltpu.async_copy(
                xt_hbm.at[lax.rem(u, _T)],
                x_v.at[pl.ds(lax.rem(u, 2) * _BATCH, _BATCH)], xsem)

        def wait_x(u):
            pltpu.make_async_copy(
                xt_hbm.at[0],
                x_v.at[pl.ds(lax.rem(u, 2) * _BATCH, _BATCH)], xsem).wait()

        def stage_slice(dt):
            pltpu.async_copy(
                tabr_hbm.at[dt],
                slice_v.at[pl.ds(lax.rem(dt, 2) * 8000, 8000)], ssem)

        def wait_slice(dt):
            pltpu.make_async_copy(
                tabr_hbm.at[0],
                slice_v.at[pl.ds(lax.rem(dt, 2) * 8000, 8000)], ssem).wait()

        def wait_put(slot):
            pltpu.make_async_copy(
                blk_v.at[slot],
                out_hbm.at[0, 0, pl.ds(0, 16)], osem).wait()

        stage_x(lo)
        stage_slice(lo // _T)

        def u_body(u, _):
            dt = u // _T
            t = lax.rem(u, _T)
            xbase = lax.rem(u, 2) * _BATCH
            soff = jnp.full((16,), lax.rem(dt, 2) * 8000, jnp.int32)
            wait_x(u)

            @pl.when(u + 1 < hi)
            def _():
                stage_x(u + 1)

            @pl.when(jnp.logical_or(u == lo, t == 0))
            def _():
                wait_slice(dt)

            @pl.when(jnp.logical_and(u + 1 < hi, lax.rem(u + 1, _T) == 0))
            def _():
                stage_slice((u + 1) // _T)

            for h in range(2):
                g = (u - lo) * 2 + h

                @pl.when(g >= 2)
                def _():
                    wait_put(h)

                @plsc.parallel_loop(0, 16, unroll=2)
                def bt_body(btl):
                    for bl in range(8):
                        xv = x_v[pl.ds(xbase + h * 2048 + btl * 128 + bl * 16, 16)]
                        xs = xv + soff
                        for ds in range(8):
                            vals = plsc.load_gather(slice_v, [xs + ds_off[ds]])
                            blk_v[h, btl, ds, pl.ds(bl * 16, 16)] = vals

                pltpu.async_copy(
                    blk_v.at[h],
                    out_hbm.at[t, dt, pl.ds(h * 16, 16)], osem)
            return 0

        lax.fori_loop(lo, hi, u_body, 0)
        wait_put(0)
        wait_put(1)

    return emb


_emb = _build()


def kernel(x, table):
    xt = x.T.astype(jnp.int32)
    tabr = table.T.reshape(_NDT, 8000)
    out5 = _emb(xt, tabr)
    return jnp.transpose(out5, (2, 4, 0, 1, 3)).reshape(_BATCH, _T, _NDT * 8)


# speedup vs baseline: 1.0560x; 1.0560x over previous
import functools

import jax
import jax.numpy as jnp
from jax import lax
from jax.experimental import pallas as pl
from jax.experimental.pallas import tpu as pltpu
from jax.experimental.pallas import tpu_sc as plsc

_T = 20
_NDT = 125
_NBT = 32
_BATCH = 4096


def _build():
    info = plsc.get_sparse_core_info()
    nc = info.num_cores
    nw = nc * info.num_subcores
    mesh = plsc.VectorSubcoreMesh(core_axis_name="c", subcore_axis_name="s")

    @functools.partial(
        pl.kernel,
        mesh=mesh,
        out_type=jax.ShapeDtypeStruct((_T, _NDT, _NBT, 8, 128), jnp.float32),
        scratch_types=[
            pltpu.VMEM((_T, _BATCH), jnp.int32),
            pltpu.VMEM((8000,), jnp.float32),
            pltpu.VMEM((2, 16, 8, 128), jnp.float32),
            pltpu.SemaphoreType.DMA,
            pltpu.SemaphoreType.DMA,
        ],
        compiler_params=pltpu.CompilerParams(
            use_tc_tiling_on_sc=False, needs_layout_passes=False),
    )
    def emb(xt_hbm, tabr_hbm, out_hbm, x_v, slice_v, blk_v, ssem, osem):
        wid = lax.axis_index("s") * nc + lax.axis_index("c")
        lo = wid * _NDT // nw
        hi = (wid + 1) * _NDT // nw
        pltpu.sync_copy(xt_hbm, x_v)

        ds_off = [jnp.full((16,), ds * 1000, jnp.int32) for ds in range(8)]

        def wait_put(slot):
            pltpu.make_async_copy(
                blk_v.at[slot],
                out_hbm.at[0, 0, pl.ds(0, 16)], osem).wait()

        def dt_body(dt, _):
            i = dt - lo
            pltpu.async_copy(tabr_hbm.at[dt], slice_v, ssem).wait()

            def t_body(t, _):
                for h in range(2):
                    g = (i * _T + t) * 2 + h

                    @pl.when(g >= 2)
                    def _():
                        wait_put(h)

                    @plsc.parallel_loop(0, 16, unroll=2)
                    def bt_body(btl):
                        for bl in range(8):
                            xv = x_v[t, pl.ds(h * 2048 + btl * 128 + bl * 16, 16)]
                            for ds in range(8):
                                vals = plsc.load_gather(
                                    slice_v, [xv + ds_off[ds]])
                                blk_v[h, btl, ds, pl.ds(bl * 16, 16)] = vals

                    pltpu.async_copy(
                        blk_v.at[h],
                        out_hbm.at[t, dt, pl.ds(h * 16, 16)], osem)
                return 0

            lax.fori_loop(0, _T, t_body, 0)
            return 0

        lax.fori_loop(lo, hi, dt_body, 0)
        wait_put(0)
        wait_put(1)

    return emb


_emb = _build()


def kernel(x, table):
    xt = x.T.astype(jnp.int32)
    tabr = table.T.reshape(_NDT, 8000)
    out5 = _emb(xt, tabr)
    return jnp.transpose(out5, (2, 4, 0, 1, 3)).reshape(_BATCH, _T, _NDT * 8)


# speedup vs baseline: 1.3681x; 1.2955x over previous
import functools

import jax
import jax.numpy as jnp
from jax import lax
from jax.experimental import pallas as pl
from jax.experimental.pallas import tpu as pltpu
from jax.experimental.pallas import tpu_sc as plsc

_T = 20
_NDT = 125
_NBT = 32
_BATCH = 4096
_U = 2 * _NDT


def _build():
    info = plsc.get_sparse_core_info()
    nc = info.num_cores
    nw = nc * info.num_subcores
    mesh = plsc.VectorSubcoreMesh(core_axis_name="c", subcore_axis_name="s")

    @functools.partial(
        pl.kernel,
        mesh=mesh,
        out_type=jax.ShapeDtypeStruct((_T, _NDT, _NBT, 8, 128), jnp.float32),
        scratch_types=[
            pltpu.VMEM((10, _BATCH), jnp.int32),
            pltpu.VMEM((8000,), jnp.float32),
            pltpu.VMEM((2, _NBT, 8, 128), jnp.float32),
            pltpu.SemaphoreType.DMA,
            pltpu.SemaphoreType.DMA,
        ],
        compiler_params=pltpu.CompilerParams(
            use_tc_tiling_on_sc=False, needs_layout_passes=False),
    )
    def emb(xt_hbm, tabr_hbm, out_hbm, x_v, slice_v, blk_v, ssem, osem):
        wid = lax.axis_index("s") * nc + lax.axis_index("c")
        lo = wid * _U // nw
        hi = (wid + 1) * _U // nw

        ds_off = [jnp.full((16,), ds * 1000, jnp.int32) for ds in range(8)]

        def wait_put(slot):
            pltpu.make_async_copy(
                blk_v.at[slot],
                out_hbm.at[0, 0], osem).wait()

        def u_body(u, _):
            th = u // _NDT
            dt = lax.rem(u, _NDT)

            @pl.when(jnp.logical_or(u == lo, dt == 0))
            def _():
                pltpu.sync_copy(xt_hbm.at[pl.ds(th * 10, 10)], x_v)

            pltpu.async_copy(tabr_hbm.at[dt], slice_v, ssem).wait()

            def tp_body(tp, _):
                for sl in range(2):
                    tl = tp * 2 + sl
                    t = th * 10 + tl
                    g = (u - lo) * 10 + tl

                    @pl.when(g >= 2)
                    def _():
                        wait_put(sl)

                    @plsc.parallel_loop(0, _NBT, unroll=2)
                    def bt_body(btl):
                        for bl in range(8):
                            xv = x_v[tl, pl.ds(btl * 128 + bl * 16, 16)]
                            for ds in range(8):
                                vals = plsc.load_gather(
                                    slice_v, [xv + ds_off[ds]])
                                blk_v[sl, btl, ds, pl.ds(bl * 16, 16)] = vals

                    pltpu.async_copy(
                        blk_v.at[sl],
                        out_hbm.at[t, dt], osem)
                return 0

            lax.fori_loop(0, 5, tp_body, 0)
            return 0

        lax.fori_loop(lo, hi, u_body, 0)
        wait_put(0)
        wait_put(1)

    return emb


_emb = _build()


def kernel(x, table):
    xt = x.T.astype(jnp.int32)
    tabr = table.T.reshape(_NDT, 8000)
    out5 = _emb(xt, tabr)
    return jnp.transpose(out5, (2, 4, 0, 1, 3)).reshape(_BATCH, _T, _NDT * 8)
